# baseline (speedup 1.0000x reference)
import functools

import jax
import jax.numpy as jnp
from jax import lax
from jax.experimental import pallas as pl
from jax.experimental.pallas import tpu as pltpu
from jax.experimental.pallas import tpu_sc as plsc

_B, _D, _M, _N = 16384, 256, 8192, 100000
_BB = 4096
_MC = 512
_COMMIT = 0.25

_NC = 2
_NS = 16
_NW = _NC * _NS
_BPW = _B // _NW
_GCH = 128
_OSZ = 3128


def _dist_body(x_ref, ct2_ref, iota_ref, enc_ref, stats_ref,
               xsq_s, minv_s, arg_s):
    j = pl.program_id(1)
    nj = pl.num_programs(1)

    @pl.when(j == 0)
    def _():
        x0 = x_ref[...]
        xsq_s[...] = jnp.sum(x0 * x0, axis=1, keepdims=True)

    x = x_ref[...]
    cb2 = ct2_ref[...]
    p2 = lax.dot_general(x, cb2, (((1,), (1,)), ((), ())),
                         preferred_element_type=jnp.float32)
    e_sq = 0.25 * jnp.sum(cb2 * cb2, axis=1)[None, :]
    d = (xsq_s[...] + p2) + e_sq

    @pl.when(j == 0)
    def _():
        minv_s[...] = d
        arg_s[...] = jnp.zeros((_BB, _MC), jnp.float32)

    @pl.when(j > 0)
    def _():
        acc = minv_s[...]
        lt = d < acc
        minv_s[...] = jnp.where(lt, d, acc)
        arg_s[...] = jnp.where(lt, j.astype(jnp.float32), arg_s[...])

    @pl.when(j == nj - 1)
    def _():
        acc = minv_s[...]
        m = jnp.min(acc, axis=1, keepdims=True)
        gidx = arg_s[...] * float(_MC) + iota_ref[...]
        cand = jnp.where(acc == m, gidx, 3.0e8)
        idx = jnp.min(cand, axis=1, keepdims=True)
        enc_ref[...] = idx.astype(jnp.int32)
        mm = jnp.maximum(m, 0.0)
        s0 = jnp.sum(mm)
        s1 = jnp.sum(jnp.sqrt(mm))
        lane = lax.broadcasted_iota(jnp.int32, (1, 1, 128), 2)
        stats_ref[...] = jnp.where(
            lane == 0, s0, jnp.where(lane == 1, s1, 0.0))


def _argmin_call(X_B, ct2, iota_row):
    grid = (_B // _BB, _M // _MC)
    return pl.pallas_call(
        _dist_body,
        grid=grid,
        in_specs=[
            pl.BlockSpec((_BB, _D), lambda i, j: (i, 0)),
            pl.BlockSpec((_MC, _D), lambda i, j: (j, 0)),
            pl.BlockSpec((1, _MC), lambda i, j: (0, 0)),
        ],
        out_specs=[
            pl.BlockSpec((_BB, 1), lambda i, j: (i, 0)),
            pl.BlockSpec((1, 1, 128), lambda i, j: (i, 0, 0)),
        ],
        out_shape=[
            jax.ShapeDtypeStruct((_B, 1), jnp.int32),
            jax.ShapeDtypeStruct((_B // _BB, 1, 128), jnp.float32),
        ],
        scratch_shapes=[
            pltpu.VMEM((_BB, 1), jnp.float32),
            pltpu.VMEM((_BB, _MC), jnp.float32),
            pltpu.VMEM((_BB, _MC), jnp.float32),
        ],
    )(X_B, ct2, iota_row)


def _sc_body(cb_hbm, enc_hbm, bidx_hbm, cind_hbm, q_hbm, newc_hbm,
             idx_v, rows0_v, rows1_v, bidx_v, encb_v, cloc_v, pos_v,
             bsem, gsem, osem):
    wid = lax.axis_index("s") * _NC + lax.axis_index("c")
    nch = _BPW // _GCH
    rows = [rows0_v, rows1_v]

    obase = jnp.minimum(wid * _OSZ, _N - _OSZ)
    a_bidx = pltpu.async_copy(bidx_hbm, bidx_v, bsem)
    a_enc = pltpu.async_copy(enc_hbm, encb_v, bsem)
    a_cloc = pltpu.async_copy(cind_hbm.at[pl.ds(obase, _OSZ)], cloc_v, bsem)

    pltpu.sync_copy(enc_hbm.at[pl.ds(wid * _BPW, _BPW)], idx_v)
    gathers = [pltpu.async_copy(cb_hbm.at[idx_v.at[pl.ds(k * _GCH, _GCH)]],
                                rows[k % 2], gsem)
               for k in range(2)]
    outs = []

    a_bidx.wait()
    a_enc.wait()
    a_cloc.wait()

    lanes = lax.iota(jnp.int32, 16)
    zeros16 = jnp.zeros((16,), jnp.int32)
    lanebit = jnp.left_shift(jnp.int32(1), lanes)

    def step(v, carry):
        for u in range(2):
            off = v * 32 + u * 16
            vb = bidx_v[pl.ds(off, 16)]
            ve = encb_v[pl.ds(off, 16)]
            rel = vb - obase
            owned = (rel >= 0) & (rel < _OSZ)
            relc = jnp.where(owned, rel, 0)
            plsc.store_scatter(pos_v, [relc], zeros16, mask=owned)
            plsc.addupdate_scatter(pos_v, [relc], lanebit, mask=owned)
            sums = plsc.load_gather(pos_v, [relc], mask=owned)
            hb = jnp.right_shift(
                plsc.bitcast(sums.astype(jnp.float32), jnp.int32), 23) - 127
            win = owned & (hb == lanes)
            plsc.store_scatter(cloc_v, [relc], ve, mask=win)
        return carry

    lax.fori_loop(0, _B // 32, step, 0)
    pltpu.sync_copy(cloc_v, newc_hbm.at[pl.ds(obase, _OSZ)])

    for k in range(nch):
        gathers[k].wait()
        outs.append(pltpu.async_copy(
            rows[k % 2], q_hbm.at[pl.ds(wid * _BPW + k * _GCH, _GCH)], osem))
        if k + 2 < nch:
            outs[k].wait()
            gathers.append(pltpu.async_copy(
                cb_hbm.at[idx_v.at[pl.ds((k + 2) * _GCH, _GCH)]],
                rows[k % 2], gsem))
    outs[nch - 2].wait()
    outs[nch - 1].wait()


_sc_call = functools.partial(
    pl.kernel,
    out_type=[
        jax.ShapeDtypeStruct((_B, _D), jnp.float32),
        jax.ShapeDtypeStruct((_N,), jnp.int32),
    ],
    mesh=plsc.VectorSubcoreMesh(core_axis_name="c", subcore_axis_name="s"),
    compiler_params=pltpu.CompilerParams(needs_layout_passes=False),
    scratch_types=[
        pltpu.VMEM((_BPW,), jnp.int32),
        pltpu.VMEM((_GCH, _D), jnp.float32),
        pltpu.VMEM((_GCH, _D), jnp.float32),
        pltpu.VMEM((_B,), jnp.int32),
        pltpu.VMEM((_B,), jnp.int32),
        pltpu.VMEM((_OSZ,), jnp.int32),
        pltpu.VMEM((_OSZ,), jnp.int32),
        pltpu.SemaphoreType.DMA,
        pltpu.SemaphoreType.DMA,
        pltpu.SemaphoreType.DMA,
    ],
)(_sc_body)


def kernel(X_B, batch_indices, codebook, c_indices):
    ct2 = -2.0 * codebook
    iota_row = jnp.arange(_MC, dtype=jnp.float32).reshape(1, _MC)
    enc2d, stats = _argmin_call(X_B, ct2, iota_row)
    enc = enc2d.reshape(_B)
    quantized, new_c = _sc_call(
        codebook, enc, batch_indices.astype(jnp.int32), c_indices)
    dsum = jnp.sum(stats[:, 0, 0])
    ssum = jnp.sum(stats[:, 0, 1])
    loss = _COMMIT * dsum / (_B * _D)
    vq_error = ssum / _B
    return quantized, loss, enc, new_c, vq_error


# speedup vs baseline: 1.0509x; 1.0509x over previous
import functools

import jax
import jax.numpy as jnp
from jax import lax
from jax.experimental import pallas as pl
from jax.experimental.pallas import tpu as pltpu
from jax.experimental.pallas import tpu_sc as plsc

_B, _D, _M, _N = 16384, 256, 8192, 100000
_BB = 2048
_MC = 1024
_COMMIT = 0.25

_NC = 2
_NS = 16
_NW = _NC * _NS
_BPW = _B // _NW
_GCH = 128
_OSZ = 3128


def _dist_body(x_ref, ct2_ref, iota_ref, enc_ref, stats_ref,
               xsq_s, minv_s, arg_s):
    j = pl.program_id(1)
    nj = pl.num_programs(1)

    @pl.when(j == 0)
    def _():
        x0 = x_ref[...]
        xsq_s[...] = jnp.sum(x0 * x0, axis=1, keepdims=True)

    x = x_ref[...]
    cb2 = ct2_ref[...]
    p2 = lax.dot_general(x, cb2, (((1,), (1,)), ((), ())),
                         preferred_element_type=jnp.float32)
    e_sq = 0.25 * jnp.sum(cb2 * cb2, axis=1)[None, :]
    d = (xsq_s[...] + p2) + e_sq

    @pl.when(j == 0)
    def _():
        minv_s[...] = d
        arg_s[...] = jnp.zeros((_BB, _MC), jnp.float32)

    @pl.when(j > 0)
    def _():
        acc = minv_s[...]
        arg_s[...] = jnp.where(d < acc, j.astype(jnp.float32), arg_s[...])
        minv_s[...] = jnp.minimum(d, acc)

    @pl.when(j == nj - 1)
    def _():
        acc = minv_s[...]
        m = jnp.min(acc, axis=1, keepdims=True)
        gidx = arg_s[...] * float(_MC) + iota_ref[...]
        cand = jnp.where(acc == m, gidx, 3.0e8)
        idx = jnp.min(cand, axis=1, keepdims=True)
        enc_ref[...] = idx.astype(jnp.int32)
        mm = jnp.maximum(m, 0.0)
        s0 = jnp.sum(mm)
        s1 = jnp.sum(jnp.sqrt(mm))
        lane = lax.broadcasted_iota(jnp.int32, (1, 1, 128), 2)
        stats_ref[...] = jnp.where(
            lane == 0, s0, jnp.where(lane == 1, s1, 0.0))


def _argmin_call(X_B, ct2, iota_row):
    grid = (_B // _BB, _M // _MC)
    return pl.pallas_call(
        _dist_body,
        grid=grid,
        in_specs=[
            pl.BlockSpec((_BB, _D), lambda i, j: (i, 0)),
            pl.BlockSpec((_MC, _D), lambda i, j: (j, 0)),
            pl.BlockSpec((1, _MC), lambda i, j: (0, 0)),
        ],
        out_specs=[
            pl.BlockSpec((_BB, 1), lambda i, j: (i, 0)),
            pl.BlockSpec((1, 1, 128), lambda i, j: (i, 0, 0)),
        ],
        out_shape=[
            jax.ShapeDtypeStruct((_B, 1), jnp.int32),
            jax.ShapeDtypeStruct((_B // _BB, 1, 128), jnp.float32),
        ],
        scratch_shapes=[
            pltpu.VMEM((_BB, 1), jnp.float32),
            pltpu.VMEM((_BB, _MC), jnp.float32),
            pltpu.VMEM((_BB, _MC), jnp.float32),
        ],
    )(X_B, ct2, iota_row)


def _sc_body(cb_hbm, enc_hbm, bidx_hbm, cind_hbm, q_hbm, newc_hbm,
             idx_v, rows0_v, rows1_v, bidx_v, encb_v, cloc_v, pos_v,
             bsem, gsem, osem):
    wid = lax.axis_index("s") * _NC + lax.axis_index("c")
    nch = _BPW // _GCH
    rows = [rows0_v, rows1_v]

    obase = jnp.minimum(wid * _OSZ, _N - _OSZ)
    a_bidx = pltpu.async_copy(bidx_hbm, bidx_v, bsem)
    a_enc = pltpu.async_copy(enc_hbm, encb_v, bsem)
    a_cloc = pltpu.async_copy(cind_hbm.at[pl.ds(obase, _OSZ)], cloc_v, bsem)

    pltpu.sync_copy(enc_hbm.at[pl.ds(wid * _BPW, _BPW)], idx_v)
    gathers = [pltpu.async_copy(cb_hbm.at[idx_v.at[pl.ds(k * _GCH, _GCH)]],
                                rows[k % 2], gsem)
               for k in range(2)]
    outs = []

    a_bidx.wait()
    a_enc.wait()
    a_cloc.wait()

    lanes = lax.iota(jnp.int32, 16)
    zeros16 = jnp.zeros((16,), jnp.int32)
    lanebit = jnp.left_shift(jnp.int32(1), lanes)

    def step(v, carry):
        for u in range(2):
            off = v * 32 + u * 16
            vb = bidx_v[pl.ds(off, 16)]
            ve = encb_v[pl.ds(off, 16)]
            rel = vb - obase
            owned = (rel >= 0) & (rel < _OSZ)
            relc = jnp.where(owned, rel, 0)
            plsc.store_scatter(pos_v, [relc], zeros16, mask=owned)
            plsc.addupdate_scatter(pos_v, [relc], lanebit, mask=owned)
            sums = plsc.load_gather(pos_v, [relc], mask=owned)
            hb = jnp.right_shift(
                plsc.bitcast(sums.astype(jnp.float32), jnp.int32), 23) - 127
            win = owned & (hb == lanes)
            plsc.store_scatter(cloc_v, [relc], ve, mask=win)
        return carry

    lax.fori_loop(0, _B // 32, step, 0)
    pltpu.sync_copy(cloc_v, newc_hbm.at[pl.ds(obase, _OSZ)])

    for k in range(nch):
        gathers[k].wait()
        outs.append(pltpu.async_copy(
            rows[k % 2], q_hbm.at[pl.ds(wid * _BPW + k * _GCH, _GCH)], osem))
        if k + 2 < nch:
            outs[k].wait()
            gathers.append(pltpu.async_copy(
                cb_hbm.at[idx_v.at[pl.ds((k + 2) * _GCH, _GCH)]],
                rows[k % 2], gsem))
    outs[nch - 2].wait()
    outs[nch - 1].wait()


_sc_call = functools.partial(
    pl.kernel,
    out_type=[
        jax.ShapeDtypeStruct((_B, _D), jnp.float32),
        jax.ShapeDtypeStruct((_N,), jnp.int32),
    ],
    mesh=plsc.VectorSubcoreMesh(core_axis_name="c", subcore_axis_name="s"),
    compiler_params=pltpu.CompilerParams(needs_layout_passes=False),
    scratch_types=[
        pltpu.VMEM((_BPW,), jnp.int32),
        pltpu.VMEM((_GCH, _D), jnp.float32),
        pltpu.VMEM((_GCH, _D), jnp.float32),
        pltpu.VMEM((_B,), jnp.int32),
        pltpu.VMEM((_B,), jnp.int32),
        pltpu.VMEM((_OSZ,), jnp.int32),
        pltpu.VMEM((_OSZ,), jnp.int32),
        pltpu.SemaphoreType.DMA,
        pltpu.SemaphoreType.DMA,
        pltpu.SemaphoreType.DMA,
    ],
)(_sc_body)


def kernel(X_B, batch_indices, codebook, c_indices):
    ct2 = -2.0 * codebook
    iota_row = jnp.arange(_MC, dtype=jnp.float32).reshape(1, _MC)
    enc2d, stats = _argmin_call(X_B, ct2, iota_row)
    enc = enc2d.reshape(_B)
    quantized, new_c = _sc_call(
        codebook, enc, batch_indices.astype(jnp.int32), c_indices)
    dsum = jnp.sum(stats[:, 0, 0])
    ssum = jnp.sum(stats[:, 0, 1])
    loss = _COMMIT * dsum / (_B * _D)
    vq_error = ssum / _B
    return quantized, loss, enc, new_c, vq_error
